# baseline (speedup 1.0000x reference)
import functools

import jax
import jax.numpy as jnp
from jax import lax
from jax.experimental import pallas as pl
from jax.experimental.pallas import tpu as pltpu
from jax.experimental.pallas import tpu_sc as plsc

NSEG = 512
D = 128
L = 16
NC = 2
NS = 16
NW = NC * NS
R = 400
NK = D // L
NSEGP = NSEG + 8


def _lane(vec, j):
    return jax.lax.squeeze(jax.lax.slice_in_dim(vec, j, j + 1, axis=0), (0,))


def _make_sc_pool(n_rows):
    rows_per_w = n_rows // NW
    n_chunks = rows_per_w // R
    assert rows_per_w % R == 0

    mesh = plsc.VectorSubcoreMesh(core_axis_name="c", subcore_axis_name="s")

    @functools.partial(
        pl.kernel,
        out_type=(
            jax.ShapeDtypeStruct((NW * NSEGP, 2 * D), jnp.float32),
            jax.ShapeDtypeStruct((NW, NSEG * L), jnp.float32),
        ),
        mesh=mesh,
        scratch_types=[
            pltpu.VMEM((2 * R, D), jnp.float32),
            pltpu.VMEM((2 * R,), jnp.int32),
            pltpu.VMEM((NSEGP * L,), jnp.float32),
            pltpu.VMEM((2 * D,), jnp.float32),
            pltpu.VMEM((L,), jnp.float32),
            pltpu.SMEM((1,), jnp.int32),
            pltpu.SemaphoreType.DMA((2,)),
            pltpu.SemaphoreType.DMA((2,)),
        ],
    )
    def sc_pool(x_hbm, b_hbm, out_red, out_cnt,
                xbuf, ibuf, cnttab, accbuf, cntbuf, smem,
                dsem, isem):
        cid = lax.axis_index("c")
        sid = lax.axis_index("s")
        wid = cid * NS + sid
        row0 = pl.multiple_of(wid * rows_per_w, 8)
        maxbase = pl.multiple_of(wid * NSEGP, 8)

        def x_copy(b, chunk):
            src = x_hbm.at[pl.ds(pl.multiple_of(row0 + chunk * R, 8), R), :]
            return pltpu.make_async_copy(src, xbuf.at[pl.ds(b * R, R), :],
                                         dsem.at[b])

        def i_copy(b, chunk):
            src = b_hbm.at[pl.ds(pl.multiple_of(row0 + chunk * R, 8), R)]
            return pltpu.make_async_copy(src, ibuf.at[pl.ds(b * R, R)],
                                         isem.at[b])

        for b in range(2):
            x_copy(b, b).start()
            i_copy(b, b).start()

        zero = jnp.zeros((L,), dtype=jnp.float32)
        one = jnp.ones((L,), dtype=jnp.float32)
        smem[0] = jnp.int32(NSEG)

        def init_cnt(i, _):
            cnttab[pl.ds(i * L, L)] = zero
            return 0
        lax.fori_loop(0, NSEGP, init_cnt, 0)

        def flush():
            cur = smem[0]
            cnttab[pl.ds(cur * L, L)] = cntbuf[...]
            pltpu.sync_copy(accbuf, out_red.at[maxbase + cur, :])

        def accum16(row0g, n):
            for k in range(NK):
                a = accbuf[pl.ds(k * L, L)]
                s = accbuf[pl.ds(D + k * L, L)]
                for j in range(n):
                    xv = xbuf[row0g + j, pl.ds(k * L, L)]
                    a = jnp.maximum(a, xv)
                    s = s + xv
                accbuf[pl.ds(k * L, L)] = a
                accbuf[pl.ds(D + k * L, L)] = s
            cntbuf[...] = cntbuf[...] + jnp.float32(n)

        def rows16(bvec, row0g):
            for j in range(L):
                seg = _lane(bvec, j)
                same = seg == smem[0]

                @pl.when(jnp.logical_not(same))
                def _():
                    flush()

                for k in range(NK):
                    xv = xbuf[row0g + j, pl.ds(k * L, L)]
                    a = accbuf[pl.ds(k * L, L)]
                    s = accbuf[pl.ds(D + k * L, L)]
                    accbuf[pl.ds(k * L, L)] = jnp.where(
                        same, jnp.maximum(a, xv), xv)
                    accbuf[pl.ds(D + k * L, L)] = jnp.where(
                        same, s + xv, xv)
                cntbuf[...] = jnp.where(same, cntbuf[...] + one, one)
                smem[0] = seg

        def block16(bvec, row0g):
            s_first = _lane(bvec, 0)
            s_last = _lane(bvec, L - 1)
            fast = jnp.logical_and(s_first == s_last, s_first == smem[0])

            @pl.when(fast)
            def _():
                accum16(row0g, L)

            @pl.when(jnp.logical_not(fast))
            def _():
                rows16(bvec, row0g)

        G = 2 * L

        def process(b, chunk, _):
            x_copy(b, chunk).wait()
            i_copy(b, chunk).wait()

            def group(gi, _):
                row0g = b * R + gi * G
                bv0 = ibuf[pl.ds(b * R + gi * G, L)]
                bv1 = ibuf[pl.ds(b * R + gi * G + L, L)]
                s_first = _lane(bv0, 0)
                s_last = _lane(bv1, L - 1)
                fast = jnp.logical_and(s_first == s_last, s_first == smem[0])

                @pl.when(fast)
                def _():
                    accum16(row0g, G)

                @pl.when(jnp.logical_not(fast))
                def _():
                    block16(bv0, row0g)
                    block16(bv1, row0g + L)
                return 0

            lax.fori_loop(0, R // G, group, 0)
            for t in range((R % G) // L):
                base = b * R + (R // G) * G + t * L
                block16(ibuf[pl.ds(base, L)], base)
            return 0

        def outer(i, _):
            for b in range(2):
                chunk = 2 * i + b

                @pl.when(chunk < n_chunks)
                def _():
                    process(b, chunk, 0)

                    @pl.when(chunk + 2 < n_chunks)
                    def _():
                        x_copy(b, chunk + 2).start()
                        i_copy(b, chunk + 2).start()
            return 0
        lax.fori_loop(0, (n_chunks + 1) // 2, outer, 0)
        flush()

        pltpu.sync_copy(cnttab.at[pl.ds(0, NSEG * L)], out_cnt.at[wid])

    return sc_pool


def _merge_kernel(red_ref, cnt_ref, out_ref):
    s = jnp.zeros((NSEG, D), jnp.float32)
    m = jnp.full((NSEG, D), -jnp.inf, jnp.float32)
    c = jnp.zeros((NSEG, 1), jnp.float32)
    for w in range(NW):
        cw = cnt_ref[w * NSEG:(w + 1) * NSEG, 0:1]
        valid = cw > 0
        m = jnp.maximum(
            m, jnp.where(valid, red_ref[w * NSEGP:w * NSEGP + NSEG, :D],
                         -jnp.inf))
        s = s + jnp.where(valid, red_ref[w * NSEGP:w * NSEGP + NSEG, D:],
                          0.0)
        c = c + cw
    mean = s / jnp.maximum(c, 1.0)
    m = jnp.where(c > 0, m, 0.0)
    out_ref[...] = jnp.concatenate([mean, m, s], axis=-1)


@jax.jit
def kernel(x, batch):
    n_rows = x.shape[0]
    red, cnts = _make_sc_pool(n_rows)(x, batch)
    return pl.pallas_call(
        _merge_kernel,
        out_shape=jax.ShapeDtypeStruct((NSEG, 3 * D), jnp.float32),
    )(red, cnts.reshape(NW * NSEG, L))


# speedup vs baseline: 1.3927x; 1.3927x over previous
import functools

import jax
import jax.numpy as jnp
from jax import lax
from jax.experimental import pallas as pl
from jax.experimental.pallas import tpu as pltpu
from jax.experimental.pallas import tpu_sc as plsc

NSEG = 512
D = 128
L = 16
NC = 2
NS = 16
NW = NC * NS
R = 400
NK = D // L
NSEGP = NSEG + 8


def _lane(vec, j):
    return jax.lax.squeeze(jax.lax.slice_in_dim(vec, j, j + 1, axis=0), (0,))


def _make_sc_pool(n_rows):
    rows_per_w = n_rows // NW
    n_chunks = rows_per_w // R
    assert rows_per_w % R == 0

    mesh = plsc.VectorSubcoreMesh(core_axis_name="c", subcore_axis_name="s")

    @functools.partial(
        pl.kernel,
        out_type=(
            jax.ShapeDtypeStruct((NW * NSEGP, D), jnp.float32),
            jax.ShapeDtypeStruct((NW, NSEG * L), jnp.float32),
            jax.ShapeDtypeStruct((NW * NSEGP, D), jnp.float32),
        ),
        mesh=mesh,
        scratch_types=[
            pltpu.VMEM((2 * R, D), jnp.float32),
            pltpu.VMEM((2 * R,), jnp.int32),
            pltpu.VMEM((NSEGP * L,), jnp.float32),
            pltpu.VMEM((D,), jnp.float32),
            pltpu.VMEM((D,), jnp.float32),
            pltpu.VMEM((L,), jnp.float32),
            pltpu.SMEM((1,), jnp.int32),
            pltpu.SemaphoreType.DMA((2,)),
            pltpu.SemaphoreType.DMA((2,)),
        ],
    )
    def sc_pool(x_hbm, b_hbm, out_sum, out_cnt, out_max,
                xbuf, ibuf, cnttab, accbuf, sumbuf, cntbuf, smem,
                dsem, isem):
        cid = lax.axis_index("c")
        sid = lax.axis_index("s")
        wid = cid * NS + sid
        row0 = pl.multiple_of(wid * rows_per_w, 8)
        maxbase = pl.multiple_of(wid * NSEGP, 8)

        def x_copy(b, chunk):
            src = x_hbm.at[pl.ds(pl.multiple_of(row0 + chunk * R, 8), R), :]
            return pltpu.make_async_copy(src, xbuf.at[pl.ds(b * R, R), :],
                                         dsem.at[b])

        def i_copy(b, chunk):
            src = b_hbm.at[pl.ds(pl.multiple_of(row0 + chunk * R, 8), R)]
            return pltpu.make_async_copy(src, ibuf.at[pl.ds(b * R, R)],
                                         isem.at[b])

        for b in range(2):
            x_copy(b, b).start()
            i_copy(b, b).start()

        zero = jnp.zeros((L,), dtype=jnp.float32)
        one = jnp.ones((L,), dtype=jnp.float32)
        smem[0] = jnp.int32(NSEG)

        def init_cnt(i, _):
            cnttab[pl.ds(i * L, L)] = zero
            return 0
        lax.fori_loop(0, NSEGP, init_cnt, 0)

        def _tree(vals, op):
            while len(vals) > 1:
                nxt = [op(vals[i], vals[i + 1])
                       for i in range(0, len(vals) - 1, 2)]
                if len(vals) % 2:
                    nxt.append(vals[-1])
                vals = nxt
            return vals[0]

        def flush():
            cur = smem[0]
            cnttab[pl.ds(cur * L, L)] = cntbuf[...]
            pltpu.sync_copy(accbuf, out_max.at[maxbase + cur, :])
            pltpu.sync_copy(sumbuf, out_sum.at[maxbase + cur, :])

        def process(b, chunk, _):
            x_copy(b, chunk).wait()
            i_copy(b, chunk).wait()

            def group(gi, _):
                bvec = ibuf[pl.ds(b * R + gi * L, L)]
                row0g = b * R + gi * L
                s_first = _lane(bvec, 0)
                s_last = _lane(bvec, L - 1)
                fast = jnp.logical_and(s_first == s_last, s_first == smem[0])

                @pl.when(fast)
                def _():
                    for k in range(NK):
                        xs = [xbuf[row0g + j, pl.ds(k * L, L)]
                              for j in range(L)]
                        m = _tree(xs, jnp.maximum)
                        s = _tree(xs, lambda p, q: p + q)
                        accbuf[pl.ds(k * L, L)] = jnp.maximum(
                            accbuf[pl.ds(k * L, L)], m)
                        sumbuf[pl.ds(k * L, L)] = sumbuf[pl.ds(k * L, L)] + s
                    cntbuf[...] = cntbuf[...] + jnp.float32(L)

                @pl.when(jnp.logical_not(fast))
                def _():
                    for j in range(L):
                        seg = _lane(bvec, j)
                        same = seg == smem[0]

                        @pl.when(jnp.logical_not(same))
                        def _():
                            flush()

                        for k in range(NK):
                            xv = xbuf[row0g + j, pl.ds(k * L, L)]
                            a = accbuf[pl.ds(k * L, L)]
                            s = sumbuf[pl.ds(k * L, L)]
                            accbuf[pl.ds(k * L, L)] = jnp.where(
                                same, jnp.maximum(a, xv), xv)
                            sumbuf[pl.ds(k * L, L)] = jnp.where(
                                same, s + xv, xv)
                        cntbuf[...] = jnp.where(same, cntbuf[...] + one, one)
                        smem[0] = seg
                return 0

            lax.fori_loop(0, R // L, group, 0)
            return 0

        def outer(i, _):
            for b in range(2):
                chunk = 2 * i + b
                process(b, chunk, 0)

                @pl.when(chunk + 2 < n_chunks)
                def _():
                    x_copy(b, chunk + 2).start()
                    i_copy(b, chunk + 2).start()
            return 0
        lax.fori_loop(0, n_chunks // 2, outer, 0)
        if n_chunks % 2:
            process(0, n_chunks - 1, 0)
        flush()

        pltpu.sync_copy(cnttab.at[pl.ds(0, NSEG * L)], out_cnt.at[wid])

    return sc_pool


def _merge_kernel(sum_ref, cnt_ref, max_ref, out_ref):
    cw = cnt_ref[:, :, 0:1]
    valid = cw > 0
    s = jnp.sum(jnp.where(valid, sum_ref[:, :NSEG, :], 0.0), axis=0)
    m = jnp.max(jnp.where(valid, max_ref[:, :NSEG, :], -jnp.inf), axis=0)
    c = jnp.sum(cw, axis=0)
    mean = s / jnp.maximum(c, 1.0)
    m = jnp.where(c > 0, m, 0.0)
    out_ref[...] = jnp.concatenate([mean, m, s], axis=-1)


@jax.jit
def kernel(x, batch):
    n_rows = x.shape[0]
    sums, cnts, maxs = _make_sc_pool(n_rows)(x, batch)
    return pl.pallas_call(
        _merge_kernel,
        out_shape=jax.ShapeDtypeStruct((NSEG, 3 * D), jnp.float32),
    )(sums.reshape(NW, NSEGP, D),
      cnts.reshape(NW, NSEG, L),
      maxs.reshape(NW, NSEGP, D))


# speedup vs baseline: 1.4168x; 1.0173x over previous
import functools

import jax
import jax.numpy as jnp
from jax import lax
from jax.experimental import pallas as pl
from jax.experimental.pallas import tpu as pltpu
from jax.experimental.pallas import tpu_sc as plsc

NSEG = 512
D = 128
L = 16
NC = 2
NS = 16
NW = NC * NS
R = 400
NK = D // L
NSEGP = NSEG + 8


def _lane(vec, j):
    return jax.lax.squeeze(jax.lax.slice_in_dim(vec, j, j + 1, axis=0), (0,))


def _make_sc_pool(n_rows):
    rows_per_w = n_rows // NW
    n_chunks = rows_per_w // R
    assert rows_per_w % R == 0

    mesh = plsc.VectorSubcoreMesh(core_axis_name="c", subcore_axis_name="s")

    @functools.partial(
        pl.kernel,
        out_type=(
            jax.ShapeDtypeStruct((NW * NSEGP, D), jnp.float32),
            jax.ShapeDtypeStruct((NW, NSEG * L), jnp.float32),
            jax.ShapeDtypeStruct((NW * NSEGP, D), jnp.float32),
        ),
        mesh=mesh,
        scratch_types=[
            pltpu.VMEM((2 * R, D), jnp.float32),
            pltpu.VMEM((2 * R,), jnp.int32),
            pltpu.VMEM((NSEGP * L,), jnp.float32),
            pltpu.VMEM((D,), jnp.float32),
            pltpu.VMEM((D,), jnp.float32),
            pltpu.VMEM((L,), jnp.float32),
            pltpu.SMEM((1,), jnp.int32),
            pltpu.SemaphoreType.DMA((2,)),
            pltpu.SemaphoreType.DMA((2,)),
        ],
    )
    def sc_pool(x_hbm, b_hbm, out_sum, out_cnt, out_max,
                xbuf, ibuf, cnttab, accbuf, sumbuf, cntbuf, smem,
                dsem, isem):
        cid = lax.axis_index("c")
        sid = lax.axis_index("s")
        wid = cid * NS + sid
        row0 = pl.multiple_of(wid * rows_per_w, 8)
        maxbase = pl.multiple_of(wid * NSEGP, 8)

        def x_copy(b, chunk):
            src = x_hbm.at[pl.ds(pl.multiple_of(row0 + chunk * R, 8), R), :]
            return pltpu.make_async_copy(src, xbuf.at[pl.ds(b * R, R), :],
                                         dsem.at[b])

        def i_copy(b, chunk):
            src = b_hbm.at[pl.ds(pl.multiple_of(row0 + chunk * R, 8), R)]
            return pltpu.make_async_copy(src, ibuf.at[pl.ds(b * R, R)],
                                         isem.at[b])

        for b in range(2):
            x_copy(b, b).start()
            i_copy(b, b).start()

        zero = jnp.zeros((L,), dtype=jnp.float32)
        one = jnp.ones((L,), dtype=jnp.float32)
        smem[0] = jnp.int32(NSEG)

        def init_cnt(i, _):
            cnttab[pl.ds(i * L, L)] = zero
            return 0
        lax.fori_loop(0, NSEGP, init_cnt, 0)

        def _tree(vals, op):
            while len(vals) > 1:
                nxt = [op(vals[i], vals[i + 1])
                       for i in range(0, len(vals) - 1, 2)]
                if len(vals) % 2:
                    nxt.append(vals[-1])
                vals = nxt
            return vals[0]

        def flush():
            cur = smem[0]
            cnttab[pl.ds(cur * L, L)] = cntbuf[...]
            pltpu.sync_copy(accbuf, out_max.at[maxbase + cur, :])
            pltpu.sync_copy(sumbuf, out_sum.at[maxbase + cur, :])

        def process(b, chunk, _):
            x_copy(b, chunk).wait()
            i_copy(b, chunk).wait()

            def group(gi, _):
                bvec = ibuf[pl.ds(b * R + gi * L, L)]
                row0g = b * R + gi * L
                s_first = _lane(bvec, 0)
                s_last = _lane(bvec, L - 1)
                fast = jnp.logical_and(s_first == s_last, s_first == smem[0])

                @pl.when(fast)
                def _():
                    for k in range(NK):
                        xs = [xbuf[row0g + j, pl.ds(k * L, L)]
                              for j in range(L)]
                        m = _tree(xs, jnp.maximum)
                        s = _tree(xs, lambda p, q: p + q)
                        accbuf[pl.ds(k * L, L)] = jnp.maximum(
                            accbuf[pl.ds(k * L, L)], m)
                        sumbuf[pl.ds(k * L, L)] = sumbuf[pl.ds(k * L, L)] + s
                    cntbuf[...] = cntbuf[...] + jnp.float32(L)

                @pl.when(jnp.logical_not(fast))
                def _():
                    for j in range(L):
                        seg = _lane(bvec, j)
                        same = seg == smem[0]

                        @pl.when(jnp.logical_not(same))
                        def _():
                            flush()

                        for k in range(NK):
                            xv = xbuf[row0g + j, pl.ds(k * L, L)]
                            a = accbuf[pl.ds(k * L, L)]
                            s = sumbuf[pl.ds(k * L, L)]
                            accbuf[pl.ds(k * L, L)] = jnp.where(
                                same, jnp.maximum(a, xv), xv)
                            sumbuf[pl.ds(k * L, L)] = jnp.where(
                                same, s + xv, xv)
                        cntbuf[...] = jnp.where(same, cntbuf[...] + one, one)
                        smem[0] = seg
                return 0

            lax.fori_loop(0, R // L, group, 0)
            return 0

        def outer(i, _):
            for b in range(2):
                chunk = 2 * i + b
                process(b, chunk, 0)

                @pl.when(chunk + 2 < n_chunks)
                def _():
                    x_copy(b, chunk + 2).start()
                    i_copy(b, chunk + 2).start()
            return 0
        lax.fori_loop(0, n_chunks // 2, outer, 0)
        if n_chunks % 2:
            process(0, n_chunks - 1, 0)
        flush()

        pltpu.sync_copy(cnttab.at[pl.ds(0, NSEG * L)], out_cnt.at[wid])

    return sc_pool


def _merge_kernel(sum_ref, cnt_ref, max_ref, out_ref):
    s = jnp.zeros((NSEG, D), jnp.float32)
    m = jnp.full((NSEG, D), -jnp.inf, jnp.float32)
    c = jnp.zeros((NSEG, 1), jnp.float32)
    for w in range(NW):
        cw = cnt_ref[w * NSEG:(w + 1) * NSEG, 0:1]
        valid = cw > 0
        s = s + jnp.where(valid, sum_ref[w * NSEGP:w * NSEGP + NSEG, :], 0.0)
        m = jnp.maximum(
            m, jnp.where(valid, max_ref[w * NSEGP:w * NSEGP + NSEG, :],
                         -jnp.inf))
        c = c + cw
    mean = s / jnp.maximum(c, 1.0)
    m = jnp.where(c > 0, m, 0.0)
    out_ref[...] = jnp.concatenate([mean, m, s], axis=-1)


@jax.jit
def kernel(x, batch):
    n_rows = x.shape[0]
    sums, cnts, maxs = _make_sc_pool(n_rows)(x, batch)
    return pl.pallas_call(
        _merge_kernel,
        out_shape=jax.ShapeDtypeStruct((NSEG, 3 * D), jnp.float32),
    )(sums, cnts.reshape(NW * NSEG, L), maxs)
